# Initial kernel scaffold; baseline (speedup 1.0000x reference)
import jax
import jax.numpy as jnp
from jax.experimental import pallas as pl


def kernel(x, y, epoch, memory):
    raise NotImplementedError("write your pallas kernel here")


# speedup vs baseline: 2.5520x; 2.5520x over previous
import functools

import jax
import jax.numpy as jnp
from jax import lax
from jax.experimental import pallas as pl
from jax.experimental.pallas import tpu as pltpu
from jax.experimental.pallas import tpu_sc as plsc

B = 1024
D = 64
DP = 128
N = 100000
K = 5
CHUNK = 2000
NCHUNK = N // CHUNK
MOM = 0.5
NEGF = -1e30
BIGI = 2**30

NC, NS = 2, 16
NW = NC * NS
BPW = B // NW


def _knn_body(x_ref, mem_ref, y_ref, idx_out, run_v, run_i):
    i = pl.program_id(0)
    sims = lax.dot_general(
        x_ref[...], mem_ref[...], (((1,), (1,)), ((), ())),
        preferred_element_type=jnp.float32)
    gcol = i * CHUNK + lax.broadcasted_iota(jnp.int32, (B, CHUNK), 1)
    sims = jnp.where(gcol == y_ref[...], -2.0, sims)

    s = sims
    chv, chi = [], []
    for k in range(K):
        m = jnp.max(s, axis=1, keepdims=True)
        c = jnp.min(jnp.where(s >= m, gcol, BIGI), axis=1, keepdims=True)
        chv.append(m)
        chi.append(c)
        if k < K - 1:
            s = jnp.where(gcol == c, NEGF, s)
    chv = jnp.concatenate(chv, axis=1)
    chi = jnp.concatenate(chi, axis=1)

    rv = jnp.where(i == 0, jnp.full((B, K), NEGF, jnp.float32), run_v[...])
    ri = jnp.where(i == 0, jnp.full((B, K), BIGI, jnp.int32), run_i[...])
    cat_v = jnp.concatenate([rv, chv], axis=1)
    cat_i = jnp.concatenate([ri, chi], axis=1)
    pos = lax.broadcasted_iota(jnp.int32, (B, 2 * K), 1)
    v = cat_v
    outv, outi = [], []
    for _ in range(K):
        m = jnp.max(v, axis=1, keepdims=True)
        p = jnp.min(jnp.where(v >= m, pos, BIGI), axis=1, keepdims=True)
        outv.append(m)
        outi.append(jnp.sum(jnp.where(pos == p, cat_i, 0), axis=1,
                            keepdims=True))
        v = jnp.where(pos == p, NEGF, v)
    run_v[...] = jnp.concatenate(outv, axis=1)
    run_i[...] = jnp.concatenate(outi, axis=1)

    @pl.when(i == NCHUNK - 1)
    def _():
        idx_out[...] = run_i[...]


def _knn(x, mem, y2):
    return pl.pallas_call(
        _knn_body,
        grid=(NCHUNK,),
        in_specs=[
            pl.BlockSpec((B, D), lambda i: (0, 0)),
            pl.BlockSpec((CHUNK, D), lambda i: (i, 0)),
            pl.BlockSpec((B, 1), lambda i: (0, 0)),
        ],
        out_specs=pl.BlockSpec((B, K), lambda i: (0, 0)),
        out_shape=jax.ShapeDtypeStruct((B, K), jnp.int32),
        scratch_shapes=[
            pltpu.VMEM((B, K), jnp.float32),
            pltpu.VMEM((B, K), jnp.int32),
        ],
    )(x, mem, y2)


def _lastdup_body(y2_ref, yr_ref, li_out):
    yrow = yr_ref[0:1, :]
    eq = y2_ref[...] == yrow
    jcol = lax.broadcasted_iota(jnp.int32, (B, B), 1)
    li_out[...] = jnp.max(jnp.where(eq, jcol, -1), axis=1, keepdims=True)


def _lastdup(y2, yr8):
    return pl.pallas_call(
        _lastdup_body,
        out_shape=jax.ShapeDtypeStruct((B, 1), jnp.int32),
    )(y2, yr8)


def _vrsqrt(a):
    bits = lax.bitcast_convert_type(a, jnp.int32)
    x0 = lax.bitcast_convert_type(jnp.int32(0x5F3759DF) - (bits >> 1),
                                  jnp.float32)
    for _ in range(4):
        x0 = x0 * (1.5 - 0.5 * a * x0 * x0)
    return x0


@functools.lru_cache(maxsize=None)
def _sc_kernels():
    mesh = plsc.VectorSubcoreMesh(core_axis_name="c", subcore_axis_name="s",
                                  num_cores=NC, num_subcores=NS)

    @functools.partial(
        pl.kernel,
        out_type=jax.ShapeDtypeStruct((B, DP), jnp.float32),
        mesh=mesh,
        scratch_types=[
            pltpu.VMEM((BPW,), jnp.int32),
            pltpu.VMEM((BPW,), jnp.int32),
            pltpu.VMEM((BPW, DP), jnp.float32),
            pltpu.VMEM((BPW, DP), jnp.float32),
            pltpu.SemaphoreType.DMA,
            pltpu.SemaphoreType.DMA,
        ],
        compiler_params=pltpu.CompilerParams(needs_layout_passes=False),
    )
    def _update(mem_hbm, x_hbm, y_hbm, li_hbm, upd_hbm, yv, liv, mrows,
                xrows, sem1, sem2):
        wid = lax.axis_index("s") * NC + lax.axis_index("c")
        base = wid * BPW
        pltpu.sync_copy(y_hbm.at[pl.ds(base, BPW)], yv)
        pltpu.sync_copy(li_hbm.at[pl.ds(base, BPW)], liv)
        cp1 = pltpu.async_copy(mem_hbm.at[yv], mrows, sem1)
        cp2 = pltpu.async_copy(x_hbm.at[liv], xrows, sem2)
        cp1.wait()
        cp2.wait()
        for r in range(BPW):
            w = [mrows[r, pl.ds(16 * c, 16)] * MOM
                 + xrows[r, pl.ds(16 * c, 16)] * (1.0 - MOM)
                 for c in range(D // 16)]
            acc = w[0] * w[0]
            for c in range(1, D // 16):
                acc = acc + w[c] * w[c]
            tot = jnp.full((16,), jnp.sum(acc), jnp.float32)
            rinv = _vrsqrt(tot)
            for c in range(D // 16):
                mrows[r, pl.ds(16 * c, 16)] = w[c] * rinv
        pltpu.sync_copy(mrows, upd_hbm.at[pl.ds(base, BPW)])

    @functools.partial(
        pl.kernel,
        out_type=(),
        mesh=mesh,
        scratch_types=[
            pltpu.VMEM((BPW,), jnp.int32),
            pltpu.VMEM((BPW, DP), jnp.float32),
            pltpu.SemaphoreType.DMA,
        ],
        compiler_params=pltpu.CompilerParams(needs_layout_passes=False),
    )
    def _scatter(upd_hbm, y_hbm, mem_ref, yv, rows, sem):
        wid = lax.axis_index("s") * NC + lax.axis_index("c")
        base = wid * BPW
        pltpu.sync_copy(y_hbm.at[pl.ds(base, BPW)], yv)
        pltpu.sync_copy(upd_hbm.at[pl.ds(base, BPW)], rows)
        pltpu.async_copy(rows, mem_ref.at[yv], sem).wait()

    return _update, _scatter


def kernel(x, y, epoch, memory):
    x = lax.stop_gradient(x)
    y = y.astype(jnp.int32)
    y2 = y.reshape(B, 1)
    yr8 = jnp.broadcast_to(y.reshape(1, B), (8, B))

    top5 = _knn(x, memory, y2)
    li = _lastdup(y2, yr8).reshape(B)

    _update, _scatter = _sc_kernels()
    mem_pad = jnp.pad(memory, ((0, 0), (0, DP - D)))
    x_pad = jnp.pad(x, ((0, 0), (0, DP - D)))
    upd = _update(mem_pad, x_pad, y, li)

    mem_ref = jax.new_ref(mem_pad)
    _scatter(upd, y, mem_ref)
    new_memory = mem_ref[...][:, :D]

    nn_idx = jnp.concatenate([y2, top5], axis=1)
    out_idx = jnp.where(epoch <= 20, jnp.broadcast_to(y2, nn_idx.shape),
                        nn_idx)
    return (out_idx, new_memory)


# speedup vs baseline: 3.9933x; 1.5648x over previous
import functools

import jax
import jax.numpy as jnp
from jax import lax
from jax.experimental import pallas as pl
from jax.experimental.pallas import tpu as pltpu
from jax.experimental.pallas import tpu_sc as plsc

B = 1024
D = 64
DP = 128
N = 100000
K = 5
CHUNK = 5000
NCHUNK = N // CHUNK
MOM = 0.5
NEGF = -1e30
BIGI = 2**30
BIGF = 3e38

NC, NS = 2, 16
NW = NC * NS
BPW = B // NW


def _knn_body(x_ref, mem_ref, y_ref, idx_out, run_v, run_i):
    i = pl.program_id(0)
    sims = lax.dot_general(
        x_ref[...], mem_ref[...], (((1,), (1,)), ((), ())),
        preferred_element_type=jnp.float32)
    lcolf = lax.broadcasted_iota(jnp.int32, (B, CHUNK), 1).astype(jnp.float32)
    yloc = (y_ref[...] - i * CHUNK).astype(jnp.float32)
    sims = jnp.where(lcolf == yloc, -2.0, sims)

    s = sims
    chv, chi = [], []
    for k in range(K):
        m = jnp.max(s, axis=1, keepdims=True)
        c = jnp.min(jnp.where(s >= m, lcolf, BIGF), axis=1, keepdims=True)
        chv.append(m)
        chi.append(c)
        if k < K - 1:
            s = jnp.where(lcolf == c, NEGF, s)
    chv = jnp.concatenate(chv, axis=1)
    chi = jnp.concatenate(chi, axis=1) + jnp.float32(i * CHUNK)

    rv = jnp.where(i == 0, jnp.full((B, K), NEGF, jnp.float32), run_v[...])
    ri = jnp.where(i == 0, jnp.full((B, K), BIGF, jnp.float32), run_i[...])
    cat_v = jnp.concatenate([rv, chv], axis=1)
    cat_i = jnp.concatenate([ri, chi], axis=1)
    v = cat_v
    outv, outi = [], []
    for _ in range(K):
        m = jnp.max(v, axis=1, keepdims=True)
        p = jnp.min(jnp.where(v >= m, cat_i, BIGF), axis=1, keepdims=True)
        outv.append(m)
        outi.append(p)
        v = jnp.where(cat_i == p, NEGF, v)
    run_v[...] = jnp.concatenate(outv, axis=1)
    run_i[...] = jnp.concatenate(outi, axis=1)

    @pl.when(i == NCHUNK - 1)
    def _():
        idx_out[...] = run_i[...].astype(jnp.int32)


def _knn(x, mem, y2):
    return pl.pallas_call(
        _knn_body,
        grid=(NCHUNK,),
        in_specs=[
            pl.BlockSpec((B, D), lambda i: (0, 0)),
            pl.BlockSpec((CHUNK, D), lambda i: (i, 0)),
            pl.BlockSpec((B, 1), lambda i: (0, 0)),
        ],
        out_specs=pl.BlockSpec((B, K), lambda i: (0, 0)),
        out_shape=jax.ShapeDtypeStruct((B, K), jnp.int32),
        scratch_shapes=[
            pltpu.VMEM((B, K), jnp.float32),
            pltpu.VMEM((B, K), jnp.float32),
        ],
    )(x, mem, y2)


def _lastdup_body(y2_ref, yr_ref, li_out):
    yrow = yr_ref[0:1, :]
    eq = y2_ref[...] == yrow
    jcol = lax.broadcasted_iota(jnp.int32, (B, B), 1)
    li_out[...] = jnp.max(jnp.where(eq, jcol, -1), axis=1, keepdims=True)


def _lastdup(y2, yr8):
    return pl.pallas_call(
        _lastdup_body,
        out_shape=jax.ShapeDtypeStruct((B, 1), jnp.int32),
    )(y2, yr8)


def _vrsqrt(a):
    bits = lax.bitcast_convert_type(a, jnp.int32)
    x0 = lax.bitcast_convert_type(jnp.int32(0x5F3759DF) - (bits >> 1),
                                  jnp.float32)
    for _ in range(4):
        x0 = x0 * (1.5 - 0.5 * a * x0 * x0)
    return x0


@functools.lru_cache(maxsize=None)
def _sc_kernels():
    mesh = plsc.VectorSubcoreMesh(core_axis_name="c", subcore_axis_name="s",
                                  num_cores=NC, num_subcores=NS)

    @functools.partial(
        pl.kernel,
        out_type=jax.ShapeDtypeStruct((B, DP), jnp.float32),
        mesh=mesh,
        scratch_types=[
            pltpu.VMEM((BPW,), jnp.int32),
            pltpu.VMEM((BPW,), jnp.int32),
            pltpu.VMEM((BPW, DP), jnp.float32),
            pltpu.VMEM((BPW, DP), jnp.float32),
            pltpu.SemaphoreType.DMA,
            pltpu.SemaphoreType.DMA,
        ],
        compiler_params=pltpu.CompilerParams(needs_layout_passes=False),
    )
    def _update(mem_hbm, x_hbm, y_hbm, li_hbm, upd_hbm, yv, liv, mrows,
                xrows, sem1, sem2):
        wid = lax.axis_index("s") * NC + lax.axis_index("c")
        base = wid * BPW
        pltpu.sync_copy(y_hbm.at[pl.ds(base, BPW)], yv)
        pltpu.sync_copy(li_hbm.at[pl.ds(base, BPW)], liv)
        cp1 = pltpu.async_copy(mem_hbm.at[yv], mrows, sem1)
        cp2 = pltpu.async_copy(x_hbm.at[liv], xrows, sem2)
        cp1.wait()
        cp2.wait()
        for r in range(BPW):
            w = [mrows[r, pl.ds(16 * c, 16)] * MOM
                 + xrows[r, pl.ds(16 * c, 16)] * (1.0 - MOM)
                 for c in range(D // 16)]
            acc = w[0] * w[0]
            for c in range(1, D // 16):
                acc = acc + w[c] * w[c]
            tot = jnp.full((16,), jnp.sum(acc), jnp.float32)
            rinv = _vrsqrt(tot)
            for c in range(D // 16):
                mrows[r, pl.ds(16 * c, 16)] = w[c] * rinv
        pltpu.sync_copy(mrows, upd_hbm.at[pl.ds(base, BPW)])

    @functools.partial(
        pl.kernel,
        out_type=(),
        mesh=mesh,
        scratch_types=[
            pltpu.VMEM((BPW,), jnp.int32),
            pltpu.VMEM((BPW, DP), jnp.float32),
            pltpu.SemaphoreType.DMA,
        ],
        compiler_params=pltpu.CompilerParams(needs_layout_passes=False),
    )
    def _scatter(upd_hbm, y_hbm, mem_ref, yv, rows, sem):
        wid = lax.axis_index("s") * NC + lax.axis_index("c")
        base = wid * BPW
        pltpu.sync_copy(y_hbm.at[pl.ds(base, BPW)], yv)
        pltpu.sync_copy(upd_hbm.at[pl.ds(base, BPW)], rows)
        pltpu.async_copy(rows, mem_ref.at[yv], sem).wait()

    return _update, _scatter


def kernel(x, y, epoch, memory):
    x = lax.stop_gradient(x)
    y = y.astype(jnp.int32)
    y2 = y.reshape(B, 1)
    yr8 = jnp.broadcast_to(y.reshape(1, B), (8, B))

    top5 = _knn(x, memory, y2)
    li = _lastdup(y2, yr8).reshape(B)

    _update, _scatter = _sc_kernels()
    mem_pad = jnp.pad(memory, ((0, 0), (0, DP - D)))
    x_pad = jnp.pad(x, ((0, 0), (0, DP - D)))
    mem_ref = jax.new_ref(mem_pad)
    upd = _update(mem_ref, x_pad, y, li)
    _scatter(upd, y, mem_ref)
    new_memory = mem_ref[...][:, :D]

    nn_idx = jnp.concatenate([y2, top5], axis=1)
    out_idx = jnp.where(epoch <= 20, jnp.broadcast_to(y2, nn_idx.shape),
                        nn_idx)
    return (out_idx, new_memory)


# speedup vs baseline: 4.6672x; 1.1688x over previous
import functools

import jax
import jax.numpy as jnp
from jax import lax
from jax.experimental import pallas as pl
from jax.experimental.pallas import tpu as pltpu
from jax.experimental.pallas import tpu_sc as plsc

B = 1024
D = 64
DP = 128
N = 100000
K = 5
CHUNK = 4096
NCHUNK = -(-N // CHUNK)
NGRP = CHUNK // 128
MOM = 0.5
NEGF = -1e30
BIGI = 2**30
BIGF = 3e38

NC, NS = 2, 16
NW = NC * NS
BPW = B // NW


def _knn_body(x_ref, mem_ref, y_ref, idx_out, run_v, run_i):
    i = pl.program_id(0)
    s = lax.dot_general(
        x_ref[...], mem_ref[...], (((1,), (1,)), ((), ())),
        preferred_element_type=jnp.float32)
    lanef = lax.broadcasted_iota(jnp.int32, (B, 128), 1).astype(jnp.float32)
    glanef = lax.broadcasted_iota(jnp.int32, (B, NGRP), 1).astype(jnp.float32)
    validf = jnp.float32(N) - jnp.float32(CHUNK) * i.astype(jnp.float32)

    grp = [s[:, 128 * j:128 * (j + 1)] for j in range(NGRP)]
    bgrp = (N - (NCHUNK - 1) * CHUNK) // 128
    grp[bgrp] = jnp.where(lanef >= validf - jnp.float32(128 * bgrp), NEGF,
                          grp[bgrp])
    gm = jnp.concatenate(
        [jnp.max(g, axis=1, keepdims=True) for g in grp], axis=1)
    gm = jnp.where(glanef >= jnp.ceil(validf / 128.0), NEGF, gm)

    chv, chi = [], []
    prevs = []
    for k in range(K + 1):
        m = jnp.max(gm, axis=1, keepdims=True)
        gsel = jnp.min(jnp.where(gm >= m, glanef, BIGF), axis=1,
                       keepdims=True)
        wsel = grp[0]
        for j in range(1, NGRP):
            wsel = jnp.where(gsel == jnp.float32(j), grp[j], wsel)
        wcol = gsel * 128.0 + lanef
        for p in prevs:
            wsel = jnp.where(wcol == p, NEGF, wsel)
        c_lane = jnp.min(jnp.where(wsel >= m, lanef, BIGF), axis=1,
                         keepdims=True)
        c_loc = gsel * 128.0 + c_lane
        chv.append(m)
        chi.append(c_loc)
        prevs.append(c_loc)
        if k < K:
            newgm = jnp.max(jnp.where(lanef == c_lane, NEGF, wsel),
                            axis=1, keepdims=True)
            gm = jnp.where(glanef == gsel, newgm, gm)
    chv = jnp.concatenate(chv, axis=1)
    chi = jnp.concatenate(chi, axis=1) + jnp.float32(i * CHUNK)
    yf = y_ref[...].astype(jnp.float32)
    chv = jnp.where(chi == yf, NEGF, chv)

    rv = jnp.where(i == 0, jnp.full((B, K), NEGF, jnp.float32), run_v[...])
    ri = jnp.where(i == 0, jnp.full((B, K), BIGF, jnp.float32), run_i[...])
    cat_v = jnp.concatenate([rv, chv], axis=1)
    cat_i = jnp.concatenate([ri, chi], axis=1)
    v = cat_v
    outv, outi = [], []
    for _ in range(K):
        m = jnp.max(v, axis=1, keepdims=True)
        p = jnp.min(jnp.where(v >= m, cat_i, BIGF), axis=1, keepdims=True)
        outv.append(m)
        outi.append(p)
        v = jnp.where(cat_i == p, NEGF, v)
    run_v[...] = jnp.concatenate(outv, axis=1)
    run_i[...] = jnp.concatenate(outi, axis=1)

    @pl.when(i == NCHUNK - 1)
    def _():
        idx_out[...] = run_i[...].astype(jnp.int32)


def _knn(x, mem, y2):
    return pl.pallas_call(
        _knn_body,
        grid=(NCHUNK,),
        in_specs=[
            pl.BlockSpec((B, D), lambda i: (0, 0)),
            pl.BlockSpec((CHUNK, D), lambda i: (i, 0)),
            pl.BlockSpec((B, 1), lambda i: (0, 0)),
        ],
        out_specs=pl.BlockSpec((B, K), lambda i: (0, 0)),
        out_shape=jax.ShapeDtypeStruct((B, K), jnp.int32),
        scratch_shapes=[
            pltpu.VMEM((B, K), jnp.float32),
            pltpu.VMEM((B, K), jnp.float32),
        ],
    )(x, mem, y2)


def _lastdup_body(y2_ref, yr_ref, li_out):
    yrow = yr_ref[0:1, :]
    eq = y2_ref[...] == yrow
    jcol = lax.broadcasted_iota(jnp.int32, (B, B), 1)
    li_out[...] = jnp.max(jnp.where(eq, jcol, -1), axis=1, keepdims=True)


def _lastdup(y2, yr8):
    return pl.pallas_call(
        _lastdup_body,
        out_shape=jax.ShapeDtypeStruct((B, 1), jnp.int32),
    )(y2, yr8)


def _vrsqrt(a):
    bits = lax.bitcast_convert_type(a, jnp.int32)
    x0 = lax.bitcast_convert_type(jnp.int32(0x5F3759DF) - (bits >> 1),
                                  jnp.float32)
    for _ in range(4):
        x0 = x0 * (1.5 - 0.5 * a * x0 * x0)
    return x0


@functools.lru_cache(maxsize=None)
def _sc_kernels():
    mesh = plsc.VectorSubcoreMesh(core_axis_name="c", subcore_axis_name="s",
                                  num_cores=NC, num_subcores=NS)

    @functools.partial(
        pl.kernel,
        out_type=jax.ShapeDtypeStruct((B, DP), jnp.float32),
        mesh=mesh,
        scratch_types=[
            pltpu.VMEM((BPW,), jnp.int32),
            pltpu.VMEM((BPW,), jnp.int32),
            pltpu.VMEM((BPW, DP), jnp.float32),
            pltpu.VMEM((BPW, DP), jnp.float32),
            pltpu.SemaphoreType.DMA,
            pltpu.SemaphoreType.DMA,
        ],
        compiler_params=pltpu.CompilerParams(needs_layout_passes=False),
    )
    def _update(mem_hbm, x_hbm, y_hbm, li_hbm, upd_hbm, yv, liv, mrows,
                xrows, sem1, sem2):
        wid = lax.axis_index("s") * NC + lax.axis_index("c")
        base = wid * BPW
        pltpu.sync_copy(y_hbm.at[pl.ds(base, BPW)], yv)
        pltpu.sync_copy(li_hbm.at[pl.ds(base, BPW)], liv)
        cp1 = pltpu.async_copy(mem_hbm.at[yv], mrows, sem1)
        cp2 = pltpu.async_copy(x_hbm.at[liv], xrows, sem2)
        cp1.wait()
        cp2.wait()
        for r in range(BPW):
            w = [mrows[r, pl.ds(16 * c, 16)] * MOM
                 + xrows[r, pl.ds(16 * c, 16)] * (1.0 - MOM)
                 for c in range(D // 16)]
            acc = w[0] * w[0]
            for c in range(1, D // 16):
                acc = acc + w[c] * w[c]
            tot = jnp.full((16,), jnp.sum(acc), jnp.float32)
            rinv = _vrsqrt(tot)
            for c in range(D // 16):
                mrows[r, pl.ds(16 * c, 16)] = w[c] * rinv
        pltpu.sync_copy(mrows, upd_hbm.at[pl.ds(base, BPW)])

    @functools.partial(
        pl.kernel,
        out_type=(),
        mesh=mesh,
        scratch_types=[
            pltpu.VMEM((BPW,), jnp.int32),
            pltpu.VMEM((BPW, DP), jnp.float32),
            pltpu.SemaphoreType.DMA,
        ],
        compiler_params=pltpu.CompilerParams(needs_layout_passes=False),
    )
    def _scatter(upd_hbm, y_hbm, mem_ref, yv, rows, sem):
        wid = lax.axis_index("s") * NC + lax.axis_index("c")
        base = wid * BPW
        pltpu.sync_copy(y_hbm.at[pl.ds(base, BPW)], yv)
        pltpu.sync_copy(upd_hbm.at[pl.ds(base, BPW)], rows)
        pltpu.async_copy(rows, mem_ref.at[yv], sem).wait()

    return _update, _scatter


def kernel(x, y, epoch, memory):
    x = lax.stop_gradient(x)
    y = y.astype(jnp.int32)
    y2 = y.reshape(B, 1)
    yr8 = jnp.broadcast_to(y.reshape(1, B), (8, B))

    top5 = _knn(x, memory, y2)
    li = _lastdup(y2, yr8).reshape(B)

    _update, _scatter = _sc_kernels()
    mem_pad = jnp.pad(memory, ((0, 0), (0, DP - D)))
    x_pad = jnp.pad(x, ((0, 0), (0, DP - D)))
    mem_ref = jax.new_ref(mem_pad)
    upd = _update(mem_ref, x_pad, y, li)
    _scatter(upd, y, mem_ref)
    new_memory = mem_ref[...][:, :D]

    nn_idx = jnp.concatenate([y2, top5], axis=1)
    out_idx = jnp.where(epoch <= 20, jnp.broadcast_to(y2, nn_idx.shape),
                        nn_idx)
    return (out_idx, new_memory)
